# baseline (speedup 1.0000x reference)
import functools

import jax
import jax.numpy as jnp
from jax import lax
from jax.experimental import pallas as pl
from jax.experimental.pallas import tpu as pltpu
from jax.experimental.pallas import tpu_sc as plsc

def _splits(n):
    out, lo = [], 0
    while lo < n:
        w = min(128, n - lo)
        out.append((lo, w))
        lo += w
    return out

_ROW_BLOCK = 4000


def _project_body(tt_ref, at_ref, wt_ref, bt_ref, wa_ref, ba_ref,
                  pt_ref, pa_ref):
    pt_ref[...] = jnp.dot(tt_ref[...], wt_ref[...],
                          preferred_element_type=jnp.float32,
                          precision=lax.Precision.HIGHEST) + bt_ref[...]
    pa_ref[...] = jnp.dot(at_ref[...], wa_ref[...],
                          preferred_element_type=jnp.float32,
                          precision=lax.Precision.HIGHEST) + ba_ref[...]


def _project_tables(time_table, act_table, W_time, b_time, W_act, b_act):
    V, D = time_table.shape
    grid = V // _ROW_BLOCK
    row_spec = pl.BlockSpec((_ROW_BLOCK, D), lambda i: (i, 0))
    full_w = pl.BlockSpec((D, D), lambda i: (0, 0))
    full_b = pl.BlockSpec((1, D), lambda i: (0, 0))
    out_shape = jax.ShapeDtypeStruct((V, D), jnp.float32)
    return pl.pallas_call(
        _project_body,
        grid=(grid,),
        in_specs=[row_spec, row_spec, full_w, full_b, full_w, full_b],
        out_specs=[row_spec, row_spec],
        out_shape=[out_shape, out_shape],
    )(time_table, act_table, W_time, b_time.reshape(1, D),
      W_act, b_act.reshape(1, D))


def _ctx_body(ctx_ref, g_ref, a_ref, o_ref):
    c = ctx_ref[...]
    gv = c[:, 0:1]
    av = c[:, 1:2]
    n = c.shape[0]
    oh_g = (lax.broadcasted_iota(jnp.int32, (n, g_ref.shape[0]), 1)
            == gv).astype(jnp.float32)
    oh_a = (lax.broadcasted_iota(jnp.int32, (n, a_ref.shape[0]), 1)
            == av).astype(jnp.float32)
    g_emb = jnp.dot(oh_g, g_ref[...], preferred_element_type=jnp.float32,
                    precision=lax.Precision.HIGHEST)
    a_emb = jnp.dot(oh_a, a_ref[...], preferred_element_type=jnp.float32,
                    precision=lax.Precision.HIGHEST)
    o_ref[...] = jnp.concatenate([g_emb, a_emb], axis=-1)


def _ctx_embed(context_tokens, gender_table, age_table):
    n = context_tokens.shape[0]
    dg = gender_table.shape[1]
    da = age_table.shape[1]
    return pl.pallas_call(
        _ctx_body,
        out_shape=jax.ShapeDtypeStruct((n, dg + da), jnp.float32),
    )(context_tokens, gender_table, age_table)


def _sc_gather(p_time, p_act, t_idx, a_idx):
    V, D = p_time.shape
    B, L = t_idx.shape
    mesh = plsc.VectorSubcoreMesh(core_axis_name="c", subcore_axis_name="s")
    out_t = jax.ShapeDtypeStruct((B, L, D), jnp.float32)

    @functools.partial(
        pl.kernel, mesh=mesh, out_type=[out_t, out_t],
        compiler_params=pltpu.CompilerParams(use_tc_tiling_on_sc=False))
    def k(pt_hbm, pa_hbm, ti_hbm, ai_hbm, ot_hbm, oa_hbm):
        def body(ti_v, ai_v, ot_v, oa_v):
            for lo, w in _splits(L):
                pltpu.sync_copy(pt_hbm.at[ti_v.at[0, pl.ds(lo, w)]],
                                ot_v.at[0, pl.ds(lo, w)])
                pltpu.sync_copy(pa_hbm.at[ai_v.at[0, pl.ds(lo, w)]],
                                oa_v.at[0, pl.ds(lo, w)])

        pltpu.emit_pipeline(
            body,
            grid=(B,),
            in_specs=[pl.BlockSpec((1, L), lambda i: (i, 0)),
                      pl.BlockSpec((1, L), lambda i: (i, 0))],
            out_specs=[pl.BlockSpec((1, L, D), lambda i: (i, 0, 0)),
                       pl.BlockSpec((1, L, D), lambda i: (i, 0, 0))],
            core_axis_name=("c", "s"),
            dimension_semantics=(pltpu.PARALLEL,),
        )(ti_hbm, ai_hbm, ot_hbm, oa_hbm)

    return k(p_time, p_act, t_idx, a_idx)


def kernel(context_tokens, time_tokens, act_tokens, time_table, act_table,
           age_table, gender_table, W_time, b_time, W_act, b_act):
    B, L = time_tokens.shape
    D = time_table.shape[1]
    t_idx = time_tokens.astype(jnp.int32)
    a_idx = act_tokens.astype(jnp.int32)

    p_time, p_act = _project_tables(time_table, act_table,
                                    W_time, b_time, W_act, b_act)
    ctx_emb = _ctx_embed(context_tokens.astype(jnp.int32),
                         gender_table, age_table)
    t_emb, a_emb = _sc_gather(p_time, p_act, t_idx, a_idx)
    return ctx_emb, t_emb, a_emb


# speedup vs baseline: 1.1992x; 1.1992x over previous
import functools

import jax
import jax.numpy as jnp
from jax import lax
from jax.experimental import pallas as pl
from jax.experimental.pallas import tpu as pltpu
from jax.experimental.pallas import tpu_sc as plsc

_GATHER_W = 128
_STEP_K = 2
_ROW_BLOCK = 4000


def _project_body(tt_ref, at_ref, wt_ref, bt_ref, wa_ref, ba_ref,
                  pt_ref, pa_ref):
    pt_ref[...] = jnp.dot(tt_ref[...], wt_ref[...],
                          preferred_element_type=jnp.float32) + bt_ref[...]
    pa_ref[...] = jnp.dot(at_ref[...], wa_ref[...],
                          preferred_element_type=jnp.float32) + ba_ref[...]


def _project_tables(time_table, act_table, W_time, b_time, W_act, b_act):
    V, D = time_table.shape
    grid = V // _ROW_BLOCK
    row_spec = pl.BlockSpec((_ROW_BLOCK, D), lambda i: (i, 0))
    full_w = pl.BlockSpec((D, D), lambda i: (0, 0))
    full_b = pl.BlockSpec((1, D), lambda i: (0, 0))
    out_shape = jax.ShapeDtypeStruct((V, D), jnp.float32)
    return pl.pallas_call(
        _project_body,
        grid=(grid,),
        in_specs=[row_spec, row_spec, full_w, full_b, full_w, full_b],
        out_specs=[row_spec, row_spec],
        out_shape=[out_shape, out_shape],
    )(time_table, act_table, W_time, b_time.reshape(1, D),
      W_act, b_act.reshape(1, D))


def _ctx_body(ctx_ref, g_ref, a_ref, o_ref):
    c = ctx_ref[...]
    gv = c[:, 0:1]
    av = c[:, 1:2]
    n = c.shape[0]
    oh_g = (lax.broadcasted_iota(jnp.int32, (n, g_ref.shape[0]), 1)
            == gv).astype(jnp.float32)
    oh_a = (lax.broadcasted_iota(jnp.int32, (n, a_ref.shape[0]), 1)
            == av).astype(jnp.float32)
    g_emb = jnp.dot(oh_g, g_ref[...], preferred_element_type=jnp.float32,
                    precision=lax.Precision.HIGHEST)
    a_emb = jnp.dot(oh_a, a_ref[...], preferred_element_type=jnp.float32,
                    precision=lax.Precision.HIGHEST)
    o_ref[...] = jnp.concatenate([g_emb, a_emb], axis=-1)


def _ctx_embed(context_tokens, gender_table, age_table):
    n = context_tokens.shape[0]
    dg = gender_table.shape[1]
    da = age_table.shape[1]
    return pl.pallas_call(
        _ctx_body,
        out_shape=jax.ShapeDtypeStruct((n, dg + da), jnp.float32),
    )(context_tokens, gender_table, age_table)


def _sc_gather(p_time, p_act, t_idx, a_idx):
    V, D = p_time.shape
    n_rows, W = t_idx.shape
    n_idx = n_rows * W
    step_rows = _STEP_K * W
    mesh = plsc.VectorSubcoreMesh(core_axis_name="c", subcore_axis_name="s")
    out_t = jax.ShapeDtypeStruct((n_idx, D), jnp.float32)

    @functools.partial(
        pl.kernel, mesh=mesh, out_type=[out_t, out_t],
        scratch_types=[pltpu.SemaphoreType.DMA],
        compiler_params=pltpu.CompilerParams(use_tc_tiling_on_sc=False))
    def k(pt_hbm, pa_hbm, ti_hbm, ai_hbm, ot_hbm, oa_hbm, sem):
        def body(ti_v, ai_v, ot_v, oa_v):
            copies = []
            for j in range(_STEP_K):
                copies.append(pltpu.async_copy(
                    pt_hbm.at[ti_v.at[j]], ot_v.at[pl.ds(j * W, W)], sem))
                copies.append(pltpu.async_copy(
                    pa_hbm.at[ai_v.at[j]], oa_v.at[pl.ds(j * W, W)], sem))
            for c in copies:
                c.wait()

        pltpu.emit_pipeline(
            body,
            grid=(n_rows // _STEP_K,),
            in_specs=[pl.BlockSpec((_STEP_K, W), lambda i: (i, 0)),
                      pl.BlockSpec((_STEP_K, W), lambda i: (i, 0))],
            out_specs=[pl.BlockSpec((step_rows, D), lambda i: (i, 0)),
                       pl.BlockSpec((step_rows, D), lambda i: (i, 0))],
            core_axis_name=("c", "s"),
            dimension_semantics=(pltpu.PARALLEL,),
        )(ti_hbm, ai_hbm, ot_hbm, oa_hbm)

    return k(p_time, p_act, t_idx, a_idx)


def kernel(context_tokens, time_tokens, act_tokens, time_table, act_table,
           age_table, gender_table, W_time, b_time, W_act, b_act):
    B, L = time_tokens.shape
    D = time_table.shape[1]
    t_idx = time_tokens.astype(jnp.int32).reshape(B * L // _GATHER_W, _GATHER_W)
    a_idx = act_tokens.astype(jnp.int32).reshape(B * L // _GATHER_W, _GATHER_W)

    p_time, p_act = _project_tables(time_table, act_table,
                                    W_time, b_time, W_act, b_act)
    ctx_emb = _ctx_embed(context_tokens.astype(jnp.int32),
                         gender_table, age_table)
    t_flat, a_flat = _sc_gather(p_time, p_act, t_idx, a_idx)
    return ctx_emb, t_flat.reshape(B, L, D), a_flat.reshape(B, L, D)


# speedup vs baseline: 1.2424x; 1.0360x over previous
import functools

import jax
import jax.numpy as jnp
from jax import lax
from jax.experimental import pallas as pl
from jax.experimental.pallas import tpu as pltpu
from jax.experimental.pallas import tpu_sc as plsc

_GATHER_W = 128
_ROW_BLOCK = 4000


def _project_body(tt_ref, at_ref, wt_ref, bt_ref, wa_ref, ba_ref,
                  pt_ref, pa_ref):
    pt_ref[...] = jnp.dot(tt_ref[...], wt_ref[...],
                          preferred_element_type=jnp.float32) + bt_ref[...]
    pa_ref[...] = jnp.dot(at_ref[...], wa_ref[...],
                          preferred_element_type=jnp.float32) + ba_ref[...]


def _project_tables(time_table, act_table, W_time, b_time, W_act, b_act):
    V, D = time_table.shape
    grid = V // _ROW_BLOCK
    row_spec = pl.BlockSpec((_ROW_BLOCK, D), lambda i: (i, 0))
    full_w = pl.BlockSpec((D, D), lambda i: (0, 0))
    full_b = pl.BlockSpec((1, D), lambda i: (0, 0))
    out_shape = jax.ShapeDtypeStruct((V, D), jnp.float32)
    return pl.pallas_call(
        _project_body,
        grid=(grid,),
        in_specs=[row_spec, row_spec, full_w, full_b, full_w, full_b],
        out_specs=[row_spec, row_spec],
        out_shape=[out_shape, out_shape],
    )(time_table, act_table, W_time, b_time.reshape(1, D),
      W_act, b_act.reshape(1, D))


def _ctx_body(ctx_ref, g_ref, a_ref, o_ref):
    c = ctx_ref[...]
    gv = c[:, 0:1]
    av = c[:, 1:2]
    n = c.shape[0]
    oh_g = (lax.broadcasted_iota(jnp.int32, (n, g_ref.shape[0]), 1)
            == gv).astype(jnp.float32)
    oh_a = (lax.broadcasted_iota(jnp.int32, (n, a_ref.shape[0]), 1)
            == av).astype(jnp.float32)
    g_emb = jnp.dot(oh_g, g_ref[...], preferred_element_type=jnp.float32,
                    precision=lax.Precision.HIGHEST)
    a_emb = jnp.dot(oh_a, a_ref[...], preferred_element_type=jnp.float32,
                    precision=lax.Precision.HIGHEST)
    o_ref[...] = jnp.concatenate([g_emb, a_emb], axis=-1)


def _ctx_embed(context_tokens, gender_table, age_table):
    n = context_tokens.shape[0]
    dg = gender_table.shape[1]
    da = age_table.shape[1]
    return pl.pallas_call(
        _ctx_body,
        out_shape=jax.ShapeDtypeStruct((n, dg + da), jnp.float32),
    )(context_tokens, gender_table, age_table)


def _sc_gather_one(p_tab, idx, step_k):
    V, D = p_tab.shape
    n_rows, W = idx.shape
    n_idx = n_rows * W
    step_rows = step_k * W
    mesh = plsc.VectorSubcoreMesh(core_axis_name="c", subcore_axis_name="s")
    out_t = jax.ShapeDtypeStruct((n_idx, D), jnp.float32)

    @functools.partial(
        pl.kernel, mesh=mesh, out_type=out_t,
        scratch_types=[pltpu.SemaphoreType.DMA],
        compiler_params=pltpu.CompilerParams(use_tc_tiling_on_sc=False))
    def k(p_hbm, i_hbm, o_hbm, sem):
        def body(i_v, o_v):
            copies = []
            for j in range(step_k):
                copies.append(pltpu.async_copy(
                    p_hbm.at[i_v.at[j]], o_v.at[pl.ds(j * W, W)], sem))
            for c in copies:
                c.wait()

        pltpu.emit_pipeline(
            body,
            grid=(n_rows // step_k,),
            in_specs=[pl.BlockSpec((step_k, W), lambda i: (i, 0))],
            out_specs=[pl.BlockSpec((step_rows, D), lambda i: (i, 0))],
            core_axis_name=("c", "s"),
            dimension_semantics=(pltpu.PARALLEL,),
        )(i_hbm, o_hbm)

    return k(p_tab, idx)


def kernel(context_tokens, time_tokens, act_tokens, time_table, act_table,
           age_table, gender_table, W_time, b_time, W_act, b_act):
    B, L = time_tokens.shape
    D = time_table.shape[1]
    t_idx = time_tokens.astype(jnp.int32).reshape(B * L // _GATHER_W, _GATHER_W)
    a_idx = act_tokens.astype(jnp.int32).reshape(B * L // _GATHER_W, _GATHER_W)

    p_time, p_act = _project_tables(time_table, act_table,
                                    W_time, b_time, W_act, b_act)
    ctx_emb = _ctx_embed(context_tokens.astype(jnp.int32),
                         gender_table, age_table)
    t_flat = _sc_gather_one(p_time, t_idx, 2)
    a_flat = _sc_gather_one(p_act, a_idx, 2)
    return ctx_emb, t_flat.reshape(B, L, D), a_flat.reshape(B, L, D)
